# Initial kernel scaffold; baseline (speedup 1.0000x reference)
import jax
import jax.numpy as jnp
from jax.experimental import pallas as pl


def kernel(z_encoded, W):
    raise NotImplementedError("write your pallas kernel here")


# speedup vs baseline: 1.4791x; 1.4791x over previous
import functools

import jax
import jax.numpy as jnp
from jax import lax
from jax.experimental import pallas as pl
from jax.experimental.pallas import tpu as pltpu
from jax.experimental.pallas import tpu_sc as plsc

D = 256
K = 8192
NB = 16
HW = 1024
N = NB * HW
KB = 8
KC = K // KB
BETA = 0.25

SC_CORES = 2
SC_SUBCORES = 16
SC_WORKERS = SC_CORES * SC_SUBCORES
TOK_PER_W = N // SC_WORKERS
GCHUNK = 256


def _argmin_body(x_ref, x2_ref, wt_ref, w2_ref, idx_ref, loss_ref):
    b = pl.program_id(0)
    x = x_ref[...]
    x2 = x2_ref[...]
    lane = lax.broadcasted_iota(jnp.int32, (1, KC), 1)
    run_min = None
    run_idx = None
    for kb in range(KB):
        wt = wt_ref[:, kb * KC:(kb + 1) * KC]
        mm = lax.dot_general(x, wt, (((1,), (0,)), ((), ())),
                             preferred_element_type=jnp.float32)
        d = (x2 + w2_ref[kb:kb + 1, :]) - 2.0 * mm
        bmin = jnp.min(d, axis=1, keepdims=True)
        cand = jnp.where(d == bmin, lane, jnp.int32(2 ** 30))
        barg = jnp.min(cand, axis=1, keepdims=True) + kb * KC
        if kb == 0:
            run_min, run_idx = bmin, barg
        else:
            upd = bmin < run_min
            run_min = jnp.where(upd, bmin, run_min)
            run_idx = jnp.where(upd, barg, run_idx)
    idx_ref[...] = run_idx
    part = jnp.sum(run_min, axis=0, keepdims=True)
    prev = jnp.where(b == 0, jnp.zeros((1, 1), jnp.float32), loss_ref[...])
    tot = prev + part
    scale = (1.0 + BETA) / (N * D)
    loss_ref[...] = jnp.where(b == NB - 1, tot * scale, tot)


def _run_argmin(flat_x, x2, wt, w2r):
    return pl.pallas_call(
        _argmin_body,
        grid=(NB,),
        in_specs=[
            pl.BlockSpec((HW, D), lambda b: (b, 0)),
            pl.BlockSpec((HW, 1), lambda b: (b, 0)),
            pl.BlockSpec((D, K), lambda b: (0, 0)),
            pl.BlockSpec((KB, KC), lambda b: (0, 0)),
        ],
        out_specs=[
            pl.BlockSpec((HW, 1), lambda b: (b, 0)),
            pl.BlockSpec((1, 1), lambda b: (0, 0)),
        ],
        out_shape=[
            jax.ShapeDtypeStruct((N, 1), jnp.int32),
            jax.ShapeDtypeStruct((1, 1), jnp.float32),
        ],
    )(flat_x, x2, wt, w2r)


def _sc_gather_body(w_hbm, idx_hbm, zq_hbm, cnt_hbm,
                    idx_v, rows_v, ones_v, zero_v, cnt_sh, sem):
    cid = lax.axis_index("c")
    sid = lax.axis_index("s")
    wid = sid * SC_CORES + cid
    base = wid * TOK_PER_W

    def fill(i, carry):
        s = pl.ds(i * 16, 16)
        zero_v[s] = jnp.zeros((16,), jnp.float32)
        ones_v[s] = jnp.ones((16,), jnp.float32)
        return carry

    lax.fori_loop(0, TOK_PER_W // 16, fill, 0)
    pltpu.sync_copy(zero_v, cnt_sh.at[pl.ds(sid * TOK_PER_W, TOK_PER_W)])
    pltpu.sync_copy(idx_hbm.at[pl.ds(base, TOK_PER_W)], idx_v)
    plsc.subcore_barrier()
    pltpu.sync_copy(ones_v, cnt_sh.at[idx_v], add=True)
    for c in range(TOK_PER_W // GCHUNK):
        pltpu.async_copy(
            w_hbm.at[idx_v.at[pl.ds(c * GCHUNK, GCHUNK)]], rows_v, sem).wait()
        pltpu.sync_copy(rows_v, zq_hbm.at[pl.ds(base + c * GCHUNK, GCHUNK)])
    plsc.subcore_barrier()

    @pl.when(sid == 0)
    def _():
        pltpu.sync_copy(cnt_sh, cnt_hbm.at[cid])


def _run_gather(W, idx):
    mesh = plsc.VectorSubcoreMesh(core_axis_name="c", subcore_axis_name="s")
    f = pl.kernel(
        _sc_gather_body,
        out_type=[
            jax.ShapeDtypeStruct((N, D), jnp.float32),
            jax.ShapeDtypeStruct((SC_CORES, K), jnp.float32),
        ],
        mesh=mesh,
        scratch_types=[
            pltpu.VMEM((TOK_PER_W,), jnp.int32),
            pltpu.VMEM((GCHUNK, D), jnp.float32),
            pltpu.VMEM((TOK_PER_W,), jnp.float32),
            pltpu.VMEM((TOK_PER_W,), jnp.float32),
            pltpu.VMEM_SHARED((K,), jnp.float32),
            pltpu.SemaphoreType.DMA,
        ],
    )
    return f(W, idx)


def _perp_body(cnt_ref, out_ref):
    c = cnt_ref[...]
    tot = jnp.sum(c, axis=0, keepdims=True)
    p = tot * (1.0 / N)
    e = jnp.sum(p * jnp.log(p + 1e-10), axis=1, keepdims=True)
    out_ref[...] = jnp.exp(-e)


def _run_perp(cnt):
    return pl.pallas_call(
        _perp_body,
        out_shape=jax.ShapeDtypeStruct((1, 1), jnp.float32),
    )(cnt)


def kernel(z_encoded, W):
    z = jnp.transpose(z_encoded, (0, 2, 3, 1))
    flat_x = z.reshape(N, D)
    x2 = jnp.sum(flat_x ** 2, axis=1, keepdims=True)
    w2 = jnp.sum(W ** 2, axis=1)
    wt = W.T
    idx2, loss = _run_argmin(flat_x, x2, wt, w2.reshape(KB, KC))
    zq_flat, cnt = _run_gather(W, idx2.reshape(N))
    perp = _run_perp(cnt)
    z_out = jnp.transpose(zq_flat.reshape(NB, 32, 32, D), (0, 3, 1, 2))
    return z_out, loss.reshape(()), perp.reshape(())
